# Initial kernel scaffold; baseline (speedup 1.0000x reference)
import jax
import jax.numpy as jnp
from jax.experimental import pallas as pl


def kernel(x, edge_index, W, attn_l, attn_r):
    raise NotImplementedError("write your pallas kernel here")


# speedup vs baseline: 39.7740x; 39.7740x over previous
import functools

import jax
import jax.numpy as jnp
from jax import lax
from jax.experimental import pallas as pl
from jax.experimental.pallas import tpu as pltpu
from jax.experimental.pallas import tpu_sc as plsc

N_NODES = 10000
N_EDGES = 320000
IN_FEATS = 128
OUT_FEATS = 32
NUM_HEADS = 4
NEG_SLOPE = 0.2

AW = 144
NC = 2
NS = 16
L = 16
NW = NC * NS
CHUNK = 128
NPAD = 10240
ROWS_PER_TILE = NPAD // NS


ERW = 16


def _proj_body(x_ref, wcat_ref, wr_ref, a_ref, er_ref):
    xb = x_ref[...]
    a_ref[...] = jnp.dot(xb, wcat_ref[...], preferred_element_type=jnp.float32)
    er_ref[...] = jnp.dot(xb, wr_ref[...], preferred_element_type=jnp.float32)


def _project(x, wcat, wr):
    mb = 2000
    grid = (N_NODES // mb,)
    return pl.pallas_call(
        _proj_body,
        grid=grid,
        in_specs=[
            pl.BlockSpec((mb, IN_FEATS), lambda i: (i, 0)),
            pl.BlockSpec((IN_FEATS, AW), lambda i: (0, 0)),
            pl.BlockSpec((IN_FEATS, ERW), lambda i: (0, 0)),
        ],
        out_specs=[
            pl.BlockSpec((mb, AW), lambda i: (i, 0)),
            pl.BlockSpec((mb, ERW), lambda i: (i, 0)),
        ],
        out_shape=[
            jax.ShapeDtypeStruct((N_NODES, AW), jnp.float32),
            jax.ShapeDtypeStruct((N_NODES, ERW), jnp.float32),
        ],
    )(x, wcat, wr)


def _edge_body(a_hbm, src_hbm, dst_hbm, er_hbm, out_hbm,
               src_v, dst_v, rows, erbuf, gbuf, acc, sem, sem2):
    cid = lax.axis_index("c")
    sid = lax.axis_index("s")
    wid = sid * NC + cid
    iota16 = lax.iota(jnp.int32, L)

    zero16 = jnp.zeros((L,), jnp.float32)

    def _zrow(r, carry):
        for c9 in range(AW // L):
            rows[r, pl.ds(c9 * L, L)] = zero16
        return carry

    lax.fori_loop(0, CHUNK, _zrow, 0)
    base = sid * ROWS_PER_TILE
    for i in range(5):
        pltpu.sync_copy(rows, acc.at[pl.ds(base + i * CHUNK, CHUNK)])
    plsc.subcore_barrier()

    nch_total = N_EDGES // CHUNK
    nwork = nch_total // NW
    extra = nch_total - nwork * NW
    nch = nwork + jnp.where(wid < extra, 1, 0)

    def _chunk(g, carry):
        off = (g * NW + wid) * CHUNK
        pltpu.sync_copy(src_hbm.at[pl.ds(off, CHUNK)], src_v)
        pltpu.sync_copy(dst_hbm.at[pl.ds(off, CHUNK)], dst_v)
        cp_rows = pltpu.async_copy(a_hbm.at[src_v], rows, sem)
        cp_er = pltpu.async_copy(er_hbm.at[dst_v], erbuf, sem2)
        cp_rows.wait()
        cp_er.wait()

        for t in range(CHUNK // L):
            e16 = t * L + iota16
            for h in range(NUM_HEADS):
                elh = plsc.load_gather(
                    rows, [e16, jnp.full((L,), IN_FEATS + h, jnp.int32)])
                erh = plsc.load_gather(
                    erbuf, [e16, jnp.full((L,), h, jnp.int32)])
                v = elh + erh
                ge = jnp.exp(jnp.maximum(v, NEG_SLOPE * v))
                plsc.store_scatter(
                    gbuf, [e16 * NUM_HEADS + h], ge)

        def _edge(j, carry2):
            jbase = jnp.full((L,), j * NUM_HEADS, jnp.int32)
            for h in range(NUM_HEADS):
                gh = plsc.load_gather(gbuf, [jbase + h])
                for c in (2 * h, 2 * h + 1):
                    rows[j, pl.ds(c * L, L)] = rows[j, pl.ds(c * L, L)] * gh
            gtail = plsc.load_gather(gbuf, [jbase + jnp.minimum(iota16, 3)])
            gtail = jnp.where(iota16 < NUM_HEADS, gtail, 0.0)
            rows[j, pl.ds(8 * L, L)] = gtail
            return carry2

        lax.fori_loop(0, CHUNK, _edge, 0)

        pltpu.sync_copy(rows, acc.at[dst_v], add=True)
        return carry

    lax.fori_loop(0, nch, _chunk, 0)
    plsc.subcore_barrier()

    for i in range(5):
        pltpu.sync_copy(acc.at[pl.ds(base + i * CHUNK, CHUNK)], rows)
        pltpu.sync_copy(rows, out_hbm.at[cid, pl.ds(base + i * CHUNK, CHUNK)])


def _edge_pass(a, src, dst, er):
    mesh = plsc.VectorSubcoreMesh(core_axis_name="c", subcore_axis_name="s")
    f = functools.partial(
        pl.kernel,
        out_type=jax.ShapeDtypeStruct((NC, NPAD, AW), jnp.float32),
        mesh=mesh,
        compiler_params=pltpu.CompilerParams(
            use_tc_tiling_on_sc=False, needs_layout_passes=False),
        scratch_types=[
            pltpu.VMEM((CHUNK,), jnp.int32),
            pltpu.VMEM((CHUNK,), jnp.int32),
            pltpu.VMEM((CHUNK, AW), jnp.float32),
            pltpu.VMEM((CHUNK, ERW), jnp.float32),
            pltpu.VMEM((CHUNK * NUM_HEADS,), jnp.float32),
            pltpu.VMEM_SHARED((NPAD, AW), jnp.float32),
            pltpu.SemaphoreType.DMA,
            pltpu.SemaphoreType.DMA,
        ],
    )(_edge_body)
    return f(a, src, dst, er)


def _combine_body(a0_ref, a1_ref, msel_ref, o_ref):
    blk = a0_ref[0] + a1_ref[0]
    den = jnp.dot(blk, msel_ref[...], preferred_element_type=jnp.float32)
    num = blk[:, :IN_FEATS]
    o_ref[...] = jnp.where(den > 0.0, num / den, 0.0)


def _combine(acc, msel):
    mb = 2000
    grid = (N_NODES // mb,)
    return pl.pallas_call(
        _combine_body,
        grid=grid,
        in_specs=[
            pl.BlockSpec((1, mb, AW), lambda i: (0, i, 0)),
            pl.BlockSpec((1, mb, AW), lambda i: (1, i, 0)),
            pl.BlockSpec((AW, IN_FEATS), lambda i: (0, 0)),
        ],
        out_specs=pl.BlockSpec((mb, IN_FEATS), lambda i: (i, 0)),
        out_shape=jax.ShapeDtypeStruct((N_NODES, IN_FEATS), jnp.float32),
    )(acc, acc, msel)


def kernel(x, edge_index, W, attn_l, attn_r):
    wh = W.reshape(NUM_HEADS, OUT_FEATS, IN_FEATS)
    wl = jnp.einsum("hdi,hd->ih", wh, attn_l[0])
    wr = jnp.einsum("hdi,hd->ih", wh, attn_r[0])
    wcat = jnp.concatenate(
        [W.T, wl, jnp.zeros((IN_FEATS, AW - IN_FEATS - NUM_HEADS),
                            jnp.float32)], axis=1)
    wr16 = jnp.concatenate(
        [wr, jnp.zeros((IN_FEATS, ERW - NUM_HEADS), jnp.float32)], axis=1)

    col = jnp.arange(IN_FEATS) // OUT_FEATS
    msel = (jnp.arange(AW)[:, None] == (IN_FEATS + col)[None, :]
            ).astype(jnp.float32)

    src = edge_index[0]
    dst = edge_index[1]

    a, er = _project(x, wcat, wr16)
    acc = _edge_pass(a, src, dst, er)
    out = _combine(acc, msel)
    return out.reshape(N_NODES, NUM_HEADS, OUT_FEATS)


# speedup vs baseline: 44.9472x; 1.1301x over previous
import functools

import jax
import jax.numpy as jnp
from jax import lax
from jax.experimental import pallas as pl
from jax.experimental.pallas import tpu as pltpu
from jax.experimental.pallas import tpu_sc as plsc

N_NODES = 10000
N_EDGES = 320000
IN_FEATS = 128
OUT_FEATS = 32
NUM_HEADS = 4
NEG_SLOPE = 0.2

AW = 144
NC = 2
NS = 16
L = 16
NW = NC * NS
CHUNK = 80
NCH = N_EDGES // (CHUNK * NW)
NPAD = 10240
ROWS_PER_TILE = NPAD // NS


ERW = 16


def _proj_body(x_ref, wcat_ref, wr_ref, a_ref, er_ref):
    xb = x_ref[...]
    a_ref[...] = jnp.dot(xb, wcat_ref[...], preferred_element_type=jnp.float32)
    er_ref[...] = jnp.dot(xb, wr_ref[...], preferred_element_type=jnp.float32)


def _project(x, wcat, wr):
    mb = 2000
    grid = (N_NODES // mb,)
    return pl.pallas_call(
        _proj_body,
        grid=grid,
        in_specs=[
            pl.BlockSpec((mb, IN_FEATS), lambda i: (i, 0)),
            pl.BlockSpec((IN_FEATS, AW), lambda i: (0, 0)),
            pl.BlockSpec((IN_FEATS, ERW), lambda i: (0, 0)),
        ],
        out_specs=[
            pl.BlockSpec((mb, AW), lambda i: (i, 0)),
            pl.BlockSpec((mb, ERW), lambda i: (i, 0)),
        ],
        out_shape=[
            jax.ShapeDtypeStruct((N_NODES, AW), jnp.float32),
            jax.ShapeDtypeStruct((N_NODES, ERW), jnp.float32),
        ],
    )(x, wcat, wr)


def _edge_body(a_hbm, src_hbm, dst_hbm, er_hbm, out_hbm,
               src0, dst0, rows0, erb0, sr0, se0,
               src1, dst1, rows1, erb1, sr1, se1,
               gbuf, acc):
    cid = lax.axis_index("c")
    sid = lax.axis_index("s")
    wid = sid * NC + cid
    iota16 = lax.iota(jnp.int32, L)
    bufs = ((src0, dst0, rows0, erb0, sr0, se0),
            (src1, dst1, rows1, erb1, sr1, se1))

    zero16 = jnp.zeros((L,), jnp.float32)

    def _zrow(r, carry):
        for c9 in range(AW // L):
            rows0[r, pl.ds(c9 * L, L)] = zero16
        return carry

    lax.fori_loop(0, CHUNK, _zrow, 0)
    base = sid * ROWS_PER_TILE
    for i in range(ROWS_PER_TILE // CHUNK):
        pltpu.sync_copy(rows0, acc.at[pl.ds(base + i * CHUNK, CHUNK)])
    plsc.subcore_barrier()

    def _issue(p, g):
        src_v, dst_v, rows, erbuf, semr, seme = bufs[p]
        off = (g * NW + wid) * CHUNK
        pltpu.sync_copy(src_hbm.at[pl.ds(off, CHUNK)], src_v)
        pltpu.sync_copy(dst_hbm.at[pl.ds(off, CHUNK)], dst_v)
        pltpu.make_async_copy(a_hbm.at[src_v], rows, semr).start()
        pltpu.make_async_copy(er_hbm.at[dst_v], erbuf, seme).start()

    def _process(p, g):
        src_v, dst_v, rows, erbuf, semr, seme = bufs[p]
        pltpu.make_async_copy(a_hbm.at[src_v], rows, semr).wait()
        pltpu.make_async_copy(er_hbm.at[dst_v], erbuf, seme).wait()

        for t in range(CHUNK // L):
            e16 = t * L + iota16
            for h in range(NUM_HEADS):
                elh = plsc.load_gather(
                    rows, [e16, jnp.full((L,), IN_FEATS + h, jnp.int32)])
                erh = plsc.load_gather(
                    erbuf, [e16, jnp.full((L,), h, jnp.int32)])
                v = elh + erh
                ge = jnp.exp(jnp.maximum(v, NEG_SLOPE * v))
                plsc.store_scatter(
                    gbuf, [e16 * NUM_HEADS + h], ge)

        def _edge(j, carry2):
            jbase = jnp.full((L,), j * NUM_HEADS, jnp.int32)
            for h in range(NUM_HEADS):
                gh = plsc.load_gather(gbuf, [jbase + h])
                for c in (2 * h, 2 * h + 1):
                    rows[j, pl.ds(c * L, L)] = rows[j, pl.ds(c * L, L)] * gh
            gtail = plsc.load_gather(gbuf, [jbase + jnp.minimum(iota16, 3)])
            gtail = jnp.where(iota16 < NUM_HEADS, gtail, 0.0)
            rows[j, pl.ds(8 * L, L)] = gtail
            return carry2

        lax.fori_loop(0, CHUNK, _edge, 0)

        pltpu.sync_copy(rows, acc.at[dst_v], add=True)

        @pl.when(g + 2 < NCH)
        def _():
            _issue(p, g + 2)

    _issue(0, 0)
    _issue(1, 1)

    def _step(g, carry):
        @pl.when(g % 2 == 0)
        def _():
            _process(0, g)

        @pl.when(g % 2 == 1)
        def _():
            _process(1, g)
        return carry

    lax.fori_loop(0, NCH, _step, 0)
    plsc.subcore_barrier()

    for i in range(ROWS_PER_TILE // CHUNK):
        pltpu.sync_copy(acc.at[pl.ds(base + i * CHUNK, CHUNK)], rows0)
        pltpu.sync_copy(rows0, out_hbm.at[cid, pl.ds(base + i * CHUNK, CHUNK)])


def _edge_pass(a, src, dst, er):
    mesh = plsc.VectorSubcoreMesh(core_axis_name="c", subcore_axis_name="s")
    f = functools.partial(
        pl.kernel,
        out_type=jax.ShapeDtypeStruct((NC, NPAD, AW), jnp.float32),
        mesh=mesh,
        compiler_params=pltpu.CompilerParams(
            use_tc_tiling_on_sc=False, needs_layout_passes=False),
        scratch_types=(
            2 * [
                pltpu.VMEM((CHUNK,), jnp.int32),
                pltpu.VMEM((CHUNK,), jnp.int32),
                pltpu.VMEM((CHUNK, AW), jnp.float32),
                pltpu.VMEM((CHUNK, ERW), jnp.float32),
                pltpu.SemaphoreType.DMA,
                pltpu.SemaphoreType.DMA,
            ]
            + [
                pltpu.VMEM((CHUNK * NUM_HEADS,), jnp.float32),
                pltpu.VMEM_SHARED((NPAD, AW), jnp.float32),
            ]
        ),
    )(_edge_body)
    return f(a, src, dst, er)


def _combine_body(a0_ref, a1_ref, msel_ref, o_ref):
    blk = a0_ref[0] + a1_ref[0]
    den = jnp.dot(blk, msel_ref[...], preferred_element_type=jnp.float32)
    num = blk[:, :IN_FEATS]
    o_ref[...] = jnp.where(den > 0.0, num / den, 0.0)


def _combine(acc, msel):
    mb = 2000
    grid = (N_NODES // mb,)
    return pl.pallas_call(
        _combine_body,
        grid=grid,
        in_specs=[
            pl.BlockSpec((1, mb, AW), lambda i: (0, i, 0)),
            pl.BlockSpec((1, mb, AW), lambda i: (1, i, 0)),
            pl.BlockSpec((AW, IN_FEATS), lambda i: (0, 0)),
        ],
        out_specs=pl.BlockSpec((mb, IN_FEATS), lambda i: (i, 0)),
        out_shape=jax.ShapeDtypeStruct((N_NODES, IN_FEATS), jnp.float32),
    )(acc, acc, msel)


def kernel(x, edge_index, W, attn_l, attn_r):
    wh = W.reshape(NUM_HEADS, OUT_FEATS, IN_FEATS)
    wl = jnp.einsum("hdi,hd->ih", wh, attn_l[0])
    wr = jnp.einsum("hdi,hd->ih", wh, attn_r[0])
    wcat = jnp.concatenate(
        [W.T, wl, jnp.zeros((IN_FEATS, AW - IN_FEATS - NUM_HEADS),
                            jnp.float32)], axis=1)
    wr16 = jnp.concatenate(
        [wr, jnp.zeros((IN_FEATS, ERW - NUM_HEADS), jnp.float32)], axis=1)

    col = jnp.arange(IN_FEATS) // OUT_FEATS
    msel = (jnp.arange(AW)[:, None] == (IN_FEATS + col)[None, :]
            ).astype(jnp.float32)

    src = edge_index[0]
    dst = edge_index[1]

    a, er = _project(x, wcat, wr16)
    acc = _edge_pass(a, src, dst, er)
    out = _combine(acc, msel)
    return out.reshape(N_NODES, NUM_HEADS, OUT_FEATS)


# speedup vs baseline: 55.5521x; 1.2359x over previous
import functools

import jax
import jax.numpy as jnp
from jax import lax
from jax.experimental import pallas as pl
from jax.experimental.pallas import tpu as pltpu
from jax.experimental.pallas import tpu_sc as plsc

N_NODES = 10000
N_EDGES = 320000
IN_FEATS = 128
OUT_FEATS = 32
NUM_HEADS = 4
NEG_SLOPE = 0.2

AW = 144
NC = 2
NS = 16
L = 16
NW = NC * NS
CHUNK = 80
NCH = N_EDGES // (CHUNK * NW)
NPAD = 10112
ROWS_PER_TILE = NPAD // NS
NBUF = 3


ERW = 16


def _proj_body(x_ref, wcat_ref, wr_ref, a_ref, er_ref):
    xb = x_ref[...]
    a_ref[...] = jnp.dot(xb, wcat_ref[...], preferred_element_type=jnp.float32)
    er_ref[...] = jnp.dot(xb, wr_ref[...], preferred_element_type=jnp.float32)


def _project(x, wcat, wr):
    mb = 2000
    grid = (N_NODES // mb,)
    return pl.pallas_call(
        _proj_body,
        grid=grid,
        in_specs=[
            pl.BlockSpec((mb, IN_FEATS), lambda i: (i, 0)),
            pl.BlockSpec((IN_FEATS, AW), lambda i: (0, 0)),
            pl.BlockSpec((IN_FEATS, ERW), lambda i: (0, 0)),
        ],
        out_specs=[
            pl.BlockSpec((mb, AW), lambda i: (i, 0)),
            pl.BlockSpec((mb, ERW), lambda i: (i, 0)),
        ],
        out_shape=[
            jax.ShapeDtypeStruct((N_NODES, AW), jnp.float32),
            jax.ShapeDtypeStruct((N_NODES, ERW), jnp.float32),
        ],
    )(x, wcat, wr)


def _edge_body(a_hbm, edge_hbm, er_hbm, out_hbm,
               idx0, rows0, erb0, si0, sr0, se0, sw0,
               idx1, rows1, erb1, si1, sr1, se1, sw1,
               idx2, rows2, erb2, si2, sr2, se2, sw2,
               gbuf, acc):
    cid = lax.axis_index("c")
    sid = lax.axis_index("s")
    wid = sid * NC + cid
    iota16 = lax.iota(jnp.int32, L)
    bufs = ((idx0, rows0, erb0, si0, sr0, se0, sw0),
            (idx1, rows1, erb1, si1, sr1, se1, sw1),
            (idx2, rows2, erb2, si2, sr2, se2, sw2))

    zero16 = jnp.zeros((L,), jnp.float32)

    def _zrow(r, carry):
        for c9 in range(AW // L):
            rows0[r, pl.ds(c9 * L, L)] = zero16
        return carry

    lax.fori_loop(0, CHUNK, _zrow, 0)
    base = sid * ROWS_PER_TILE
    for i in range(ROWS_PER_TILE // CHUNK):
        pltpu.sync_copy(rows0, acc.at[pl.ds(base + i * CHUNK, CHUNK)])
    rem = ROWS_PER_TILE - (ROWS_PER_TILE // CHUNK) * CHUNK
    if rem:
        pltpu.sync_copy(
            rows0.at[pl.ds(0, rem)],
            acc.at[pl.ds(base + ROWS_PER_TILE - rem, rem)])
    plsc.subcore_barrier()

    def _issue(p, g):
        idxb, rows, erbuf, semi, semr, seme, _ = bufs[p]
        off = (g * NW + wid) * CHUNK
        pltpu.make_async_copy(
            edge_hbm.at[:, pl.ds(off, CHUNK)], idxb, semi).start()
        pltpu.make_async_copy(
            edge_hbm.at[:, pl.ds(off, CHUNK)], idxb, semi).wait()
        pltpu.make_async_copy(a_hbm.at[idxb.at[0]], rows, semr).start()
        pltpu.make_async_copy(er_hbm.at[idxb.at[1]], erbuf, seme).start()

    def _process(p, g):
        idxb, rows, erbuf, semi, semr, seme, semw = bufs[p]
        pltpu.make_async_copy(a_hbm.at[idxb.at[0]], rows, semr).wait()
        pltpu.make_async_copy(er_hbm.at[idxb.at[1]], erbuf, seme).wait()

        for t in range(CHUNK // L):
            e16 = t * L + iota16
            for h in range(NUM_HEADS):
                elh = plsc.load_gather(
                    rows, [e16, jnp.full((L,), IN_FEATS + h, jnp.int32)])
                erh = plsc.load_gather(
                    erbuf, [e16, jnp.full((L,), h, jnp.int32)])
                v = elh + erh
                ge = jnp.exp(jnp.maximum(v, NEG_SLOPE * v))
                plsc.store_scatter(
                    gbuf, [e16 * NUM_HEADS + h], ge)

        def _edge(j, carry2):
            jbase = jnp.full((L,), j * NUM_HEADS, jnp.int32)
            for h in range(NUM_HEADS):
                gh = plsc.load_gather(gbuf, [jbase + h])
                for c in (2 * h, 2 * h + 1):
                    rows[j, pl.ds(c * L, L)] = rows[j, pl.ds(c * L, L)] * gh
            gtail = plsc.load_gather(gbuf, [jbase + jnp.minimum(iota16, 3)])
            gtail = jnp.where(iota16 < NUM_HEADS, gtail, 0.0)
            rows[j, pl.ds(8 * L, L)] = gtail
            return carry2

        lax.fori_loop(0, CHUNK, _edge, 0)

        pltpu.make_async_copy(rows, acc.at[idxb.at[1]], semw).start(add=True)

        @pl.when(g + 2 < NCH)
        def _():
            pn = (g + 2) % NBUF
            for pi in range(NBUF):
                @pl.when(pn == pi)
                def _():
                    @pl.when(g >= 1)
                    def _():
                        bw = bufs[pi]
                        pltpu.make_async_copy(
                            bw[1], acc.at[bw[0].at[1]], bw[6]).wait()
                    _issue(pi, g + 2)

    _issue(0, 0)
    _issue(1, 1)

    def _step(g, carry):
        for pi in range(NBUF):
            @pl.when(g % NBUF == pi)
            def _():
                _process(pi, g)
        return carry

    lax.fori_loop(0, NCH, _step, 0)

    for pi in range(NBUF):
        bw = bufs[pi]
        pltpu.make_async_copy(bw[1], acc.at[bw[0].at[1]], bw[6]).wait()
    plsc.subcore_barrier()

    for i in range(ROWS_PER_TILE // CHUNK):
        pltpu.sync_copy(acc.at[pl.ds(base + i * CHUNK, CHUNK)], rows0)
        pltpu.sync_copy(rows0, out_hbm.at[cid, pl.ds(base + i * CHUNK, CHUNK)])
    if rem:
        pltpu.sync_copy(
            acc.at[pl.ds(base + ROWS_PER_TILE - rem, rem)],
            rows0.at[pl.ds(0, rem)])
        pltpu.sync_copy(
            rows0.at[pl.ds(0, rem)],
            out_hbm.at[cid, pl.ds(base + ROWS_PER_TILE - rem, rem)])


def _edge_pass(a, edge_index, er):
    mesh = plsc.VectorSubcoreMesh(core_axis_name="c", subcore_axis_name="s")
    f = functools.partial(
        pl.kernel,
        out_type=jax.ShapeDtypeStruct((NC, NPAD, AW), jnp.float32),
        mesh=mesh,
        compiler_params=pltpu.CompilerParams(
            use_tc_tiling_on_sc=False, needs_layout_passes=False),
        scratch_types=(
            NBUF * [
                pltpu.VMEM((2, CHUNK), jnp.int32),
                pltpu.VMEM((CHUNK, AW), jnp.float32),
                pltpu.VMEM((CHUNK, ERW), jnp.float32),
                pltpu.SemaphoreType.DMA,
                pltpu.SemaphoreType.DMA,
                pltpu.SemaphoreType.DMA,
                pltpu.SemaphoreType.DMA,
            ]
            + [
                pltpu.VMEM((CHUNK * NUM_HEADS,), jnp.float32),
                pltpu.VMEM_SHARED((NPAD, AW), jnp.float32),
            ]
        ),
    )(_edge_body)
    return f(a, edge_index, er)


def _combine_body(a0_ref, a1_ref, msel_ref, o_ref):
    blk = a0_ref[0] + a1_ref[0]
    den = jnp.dot(blk, msel_ref[...], preferred_element_type=jnp.float32)
    num = blk[:, :IN_FEATS]
    o_ref[...] = jnp.where(den > 0.0, num / den, 0.0)


def _combine(acc, msel):
    mb = 2000
    grid = (N_NODES // mb,)
    return pl.pallas_call(
        _combine_body,
        grid=grid,
        in_specs=[
            pl.BlockSpec((1, mb, AW), lambda i: (0, i, 0)),
            pl.BlockSpec((1, mb, AW), lambda i: (1, i, 0)),
            pl.BlockSpec((AW, IN_FEATS), lambda i: (0, 0)),
        ],
        out_specs=pl.BlockSpec((mb, IN_FEATS), lambda i: (i, 0)),
        out_shape=jax.ShapeDtypeStruct((N_NODES, IN_FEATS), jnp.float32),
    )(acc, acc, msel)


def kernel(x, edge_index, W, attn_l, attn_r):
    wh = W.reshape(NUM_HEADS, OUT_FEATS, IN_FEATS)
    wl = jnp.einsum("hdi,hd->ih", wh, attn_l[0])
    wr = jnp.einsum("hdi,hd->ih", wh, attn_r[0])
    wcat = jnp.concatenate(
        [W.T, wl, jnp.zeros((IN_FEATS, AW - IN_FEATS - NUM_HEADS),
                            jnp.float32)], axis=1)
    wr16 = jnp.concatenate(
        [wr, jnp.zeros((IN_FEATS, ERW - NUM_HEADS), jnp.float32)], axis=1)

    col = jnp.arange(IN_FEATS) // OUT_FEATS
    msel = (jnp.arange(AW)[:, None] == (IN_FEATS + col)[None, :]
            ).astype(jnp.float32)

    a, er = _project(x, wcat, wr16)
    acc = _edge_pass(a, edge_index, er)
    out = _combine(acc, msel)
    return out.reshape(N_NODES, NUM_HEADS, OUT_FEATS)


# speedup vs baseline: 94.1197x; 1.6943x over previous
import functools

import jax
import jax.numpy as jnp
from jax import lax
from jax.experimental import pallas as pl
from jax.experimental.pallas import tpu as pltpu
from jax.experimental.pallas import tpu_sc as plsc

N_NODES = 10000
N_EDGES = 320000
IN_FEATS = 128
OUT_FEATS = 32
NUM_HEADS = 4
NEG_SLOPE = 0.2

AW = 144
NC = 2
NS = 16
L = 16
NW = NC * NS
CHUNK = 80
NCH = N_EDGES // (CHUNK * NW)
NPAD = 10112
ROWS_PER_TILE = NPAD // NS
NBUF = 3


ERW = 16


def _proj_body(x_ref, wcat_ref, wr_ref, a_ref, er_ref):
    xb = x_ref[...]
    a_ref[...] = jnp.dot(xb, wcat_ref[...], preferred_element_type=jnp.float32)
    er_ref[...] = jnp.dot(xb, wr_ref[...], preferred_element_type=jnp.float32)


def _project(x, wcat, wr):
    mb = 2000
    grid = (N_NODES // mb,)
    return pl.pallas_call(
        _proj_body,
        grid=grid,
        in_specs=[
            pl.BlockSpec((mb, IN_FEATS), lambda i: (i, 0)),
            pl.BlockSpec((IN_FEATS, AW), lambda i: (0, 0)),
            pl.BlockSpec((IN_FEATS, ERW), lambda i: (0, 0)),
        ],
        out_specs=[
            pl.BlockSpec((mb, AW), lambda i: (i, 0)),
            pl.BlockSpec((mb, ERW), lambda i: (i, 0)),
        ],
        out_shape=[
            jax.ShapeDtypeStruct((N_NODES, AW), jnp.float32),
            jax.ShapeDtypeStruct((N_NODES, ERW), jnp.float32),
        ],
    )(x, wcat, wr)


def _edge_body(a_hbm, edge_hbm, er_hbm, out_hbm,
               idx0, rows0, erb0, si0, sr0, se0, sw0,
               idx1, rows1, erb1, si1, sr1, se1, sw1,
               idx2, rows2, erb2, si2, sr2, se2, sw2,
               gbuf, acc):
    cid = lax.axis_index("c")
    sid = lax.axis_index("s")
    wid = sid * NC + cid
    iota16 = lax.iota(jnp.int32, L)
    bufs = ((idx0, rows0, erb0, si0, sr0, se0, sw0),
            (idx1, rows1, erb1, si1, sr1, se1, sw1),
            (idx2, rows2, erb2, si2, sr2, se2, sw2))

    zero16 = jnp.zeros((L,), jnp.float32)

    def _zrow(r, carry):
        for c9 in range(AW // L):
            rows0[r, pl.ds(c9 * L, L)] = zero16
        return carry

    lax.fori_loop(0, CHUNK, _zrow, 0)
    base = sid * ROWS_PER_TILE
    for i in range(ROWS_PER_TILE // CHUNK):
        pltpu.sync_copy(rows0, acc.at[pl.ds(base + i * CHUNK, CHUNK)])
    rem = ROWS_PER_TILE - (ROWS_PER_TILE // CHUNK) * CHUNK
    if rem:
        pltpu.sync_copy(
            rows0.at[pl.ds(0, rem)],
            acc.at[pl.ds(base + ROWS_PER_TILE - rem, rem)])
    plsc.subcore_barrier()

    def _issue(p, g):
        idxb, rows, erbuf, semi, semr, seme, _ = bufs[p]
        off = (g * NW + wid) * CHUNK
        pltpu.make_async_copy(
            edge_hbm.at[:, pl.ds(off, CHUNK)], idxb, semi).start()
        pltpu.make_async_copy(
            edge_hbm.at[:, pl.ds(off, CHUNK)], idxb, semi).wait()
        pltpu.make_async_copy(a_hbm.at[idxb.at[0]], rows, semr).start()
        pltpu.make_async_copy(er_hbm.at[idxb.at[1]], erbuf, seme).start()

    def _process(p, g):
        idxb, rows, erbuf, semi, semr, seme, semw = bufs[p]
        pltpu.make_async_copy(a_hbm.at[idxb.at[0]], rows, semr).wait()
        pltpu.make_async_copy(er_hbm.at[idxb.at[1]], erbuf, seme).wait()

        for t in range(CHUNK // L):
            e16 = t * L + iota16
            for h in range(NUM_HEADS):
                elh = plsc.load_gather(
                    rows, [e16, jnp.full((L,), IN_FEATS + h, jnp.int32)])
                erh = plsc.load_gather(
                    erbuf, [e16, jnp.full((L,), h, jnp.int32)])
                v = elh + erh
                ge = jnp.exp(jnp.maximum(v, NEG_SLOPE * v))
                plsc.store_scatter(
                    gbuf, [e16 * NUM_HEADS + h], ge)

        @plsc.parallel_loop(0, CHUNK, unroll=4)
        def _edge(j):
            jbase = jnp.full((L,), j * NUM_HEADS, jnp.int32)
            for h in range(NUM_HEADS):
                gh = plsc.load_gather(gbuf, [jbase + h])
                for c in (2 * h, 2 * h + 1):
                    rows[j, pl.ds(c * L, L)] = rows[j, pl.ds(c * L, L)] * gh
            gtail = plsc.load_gather(gbuf, [jbase + jnp.minimum(iota16, 3)])
            gtail = jnp.where(iota16 < NUM_HEADS, gtail, 0.0)
            rows[j, pl.ds(8 * L, L)] = gtail

        pltpu.make_async_copy(rows, acc.at[idxb.at[1]], semw).start(add=True)

        @pl.when(g + 2 < NCH)
        def _():
            pn = (g + 2) % NBUF
            for pi in range(NBUF):
                @pl.when(pn == pi)
                def _():
                    @pl.when(g >= 1)
                    def _():
                        bw = bufs[pi]
                        pltpu.make_async_copy(
                            bw[1], acc.at[bw[0].at[1]], bw[6]).wait()
                    _issue(pi, g + 2)

    _issue(0, 0)
    _issue(1, 1)

    def _step(g, carry):
        for pi in range(NBUF):
            @pl.when(g % NBUF == pi)
            def _():
                _process(pi, g)
        return carry

    lax.fori_loop(0, NCH, _step, 0)

    for pi in range(NBUF):
        bw = bufs[pi]
        pltpu.make_async_copy(bw[1], acc.at[bw[0].at[1]], bw[6]).wait()
    plsc.subcore_barrier()

    for i in range(ROWS_PER_TILE // CHUNK):
        pltpu.sync_copy(acc.at[pl.ds(base + i * CHUNK, CHUNK)], rows0)
        pltpu.sync_copy(rows0, out_hbm.at[cid, pl.ds(base + i * CHUNK, CHUNK)])
    if rem:
        pltpu.sync_copy(
            acc.at[pl.ds(base + ROWS_PER_TILE - rem, rem)],
            rows0.at[pl.ds(0, rem)])
        pltpu.sync_copy(
            rows0.at[pl.ds(0, rem)],
            out_hbm.at[cid, pl.ds(base + ROWS_PER_TILE - rem, rem)])


def _edge_pass(a, edge_index, er):
    mesh = plsc.VectorSubcoreMesh(core_axis_name="c", subcore_axis_name="s")
    f = functools.partial(
        pl.kernel,
        out_type=jax.ShapeDtypeStruct((NC, NPAD, AW), jnp.float32),
        mesh=mesh,
        compiler_params=pltpu.CompilerParams(
            use_tc_tiling_on_sc=False, needs_layout_passes=False),
        scratch_types=(
            NBUF * [
                pltpu.VMEM((2, CHUNK), jnp.int32),
                pltpu.VMEM((CHUNK, AW), jnp.float32),
                pltpu.VMEM((CHUNK, ERW), jnp.float32),
                pltpu.SemaphoreType.DMA,
                pltpu.SemaphoreType.DMA,
                pltpu.SemaphoreType.DMA,
                pltpu.SemaphoreType.DMA,
            ]
            + [
                pltpu.VMEM((CHUNK * NUM_HEADS,), jnp.float32),
                pltpu.VMEM_SHARED((NPAD, AW), jnp.float32),
            ]
        ),
    )(_edge_body)
    return f(a, edge_index, er)


def _combine_body(a0_ref, a1_ref, msel_ref, o_ref):
    blk = a0_ref[0] + a1_ref[0]
    den = jnp.dot(blk, msel_ref[...], preferred_element_type=jnp.float32)
    num = blk[:, :IN_FEATS]
    o_ref[...] = jnp.where(den > 0.0, num / den, 0.0)


def _combine(acc, msel):
    mb = 2000
    grid = (N_NODES // mb,)
    return pl.pallas_call(
        _combine_body,
        grid=grid,
        in_specs=[
            pl.BlockSpec((1, mb, AW), lambda i: (0, i, 0)),
            pl.BlockSpec((1, mb, AW), lambda i: (1, i, 0)),
            pl.BlockSpec((AW, IN_FEATS), lambda i: (0, 0)),
        ],
        out_specs=pl.BlockSpec((mb, IN_FEATS), lambda i: (i, 0)),
        out_shape=jax.ShapeDtypeStruct((N_NODES, IN_FEATS), jnp.float32),
    )(acc, acc, msel)


def kernel(x, edge_index, W, attn_l, attn_r):
    wh = W.reshape(NUM_HEADS, OUT_FEATS, IN_FEATS)
    wl = jnp.einsum("hdi,hd->ih", wh, attn_l[0])
    wr = jnp.einsum("hdi,hd->ih", wh, attn_r[0])
    wcat = jnp.concatenate(
        [W.T, wl, jnp.zeros((IN_FEATS, AW - IN_FEATS - NUM_HEADS),
                            jnp.float32)], axis=1)
    wr16 = jnp.concatenate(
        [wr, jnp.zeros((IN_FEATS, ERW - NUM_HEADS), jnp.float32)], axis=1)

    col = jnp.arange(IN_FEATS) // OUT_FEATS
    msel = (jnp.arange(AW)[:, None] == (IN_FEATS + col)[None, :]
            ).astype(jnp.float32)

    a, er = _project(x, wcat, wr16)
    acc = _edge_pass(a, edge_index, er)
    out = _combine(acc, msel)
    return out.reshape(N_NODES, NUM_HEADS, OUT_FEATS)
